# baseline (speedup 1.0000x reference)
import functools

import jax
import jax.numpy as jnp
from jax import lax
from jax.experimental import pallas as pl
from jax.experimental.pallas import tpu as pltpu
from jax.experimental.pallas import tpu_sc as plsc

NNODES = 10000
NPAD = 10240
NEDGES = 320000
CH = 128
NCH = 80
EPT = NCH * CH
NW = 32
EPAD = EPT * NW
ROWS_PT = NPAD // 16
BLK = 1024
GRID = NPAD // BLK


def _sc_mesh():
    return plsc.VectorSubcoreMesh(
        core_axis_name="c", subcore_axis_name="s", num_cores=2, num_subcores=16
    )


def _make_deg():
    @functools.partial(
        pl.kernel,
        out_type=jax.ShapeDtypeStruct((2, NPAD, 16), jnp.float32),
        mesh=_sc_mesh(),
        scratch_types=[
            pltpu.VMEM((NCH, CH), jnp.int32),
            pltpu.VMEM((CH, 16), jnp.float32),
            pltpu.VMEM((CH, 16), jnp.float32),
            pltpu.VMEM_SHARED((NPAD, 16), jnp.float32),
        ],
    )
    def deg(dst_hbm, out_hbm, dst_v, ones_v, zero_v, acc_sh):
        c = lax.axis_index("c")
        s = lax.axis_index("s")
        wid = s * 2 + c

        def fill(i, _):
            ones_v[i, pl.ds(0, 16)] = jnp.full((16,), 1.0, jnp.float32)
            zero_v[i, pl.ds(0, 16)] = jnp.zeros((16,), jnp.float32)
            return _

        lax.fori_loop(0, CH, fill, None)
        row0 = s * ROWS_PT
        for b in range(ROWS_PT // CH):
            pltpu.sync_copy(zero_v, acc_sh.at[pl.ds(row0 + b * CH, CH)])
        plsc.subcore_barrier()

        pltpu.sync_copy(dst_hbm.at[wid], dst_v)

        def body(j, _):
            pltpu.sync_copy(ones_v, acc_sh.at[dst_v.at[j]], add=True)
            return _

        lax.fori_loop(0, NCH, body, None)
        plsc.subcore_barrier()
        pltpu.sync_copy(
            acc_sh.at[pl.ds(row0, ROWS_PT)], out_hbm.at[c, pl.ds(row0, ROWS_PT)]
        )

    return deg


def _make_agg(D):
    @functools.partial(
        pl.kernel,
        out_type=jax.ShapeDtypeStruct((2, NPAD, D), jnp.float32),
        mesh=_sc_mesh(),
        scratch_types=[
            pltpu.VMEM((NCH, CH), jnp.int32),
            pltpu.VMEM((NCH, CH), jnp.int32),
            pltpu.VMEM((CH, D), jnp.float32),
            pltpu.VMEM_SHARED((NPAD, D), jnp.float32),
            pltpu.SemaphoreType.DMA,
        ],
    )
    def agg(src_hbm, dst_hbm, table_hbm, out_hbm, src_v, dst_v, rows_v,
            acc_sh, sem):
        c = lax.axis_index("c")
        s = lax.axis_index("s")
        wid = s * 2 + c

        def zrow(i, _):
            for k in range(D // 16):
                rows_v[i, pl.ds(k * 16, 16)] = jnp.zeros((16,), jnp.float32)
            return _

        lax.fori_loop(0, CH, zrow, None)
        row0 = s * ROWS_PT
        for b in range(ROWS_PT // CH):
            pltpu.sync_copy(rows_v, acc_sh.at[pl.ds(row0 + b * CH, CH)])
        plsc.subcore_barrier()

        pltpu.sync_copy(src_hbm.at[wid], src_v)
        pltpu.sync_copy(dst_hbm.at[wid], dst_v)

        def body(j, _):
            pltpu.async_copy(table_hbm.at[src_v.at[j]], rows_v, sem).wait()
            pltpu.sync_copy(rows_v, acc_sh.at[dst_v.at[j]], add=True)
            return _

        lax.fori_loop(0, NCH, body, None)
        plsc.subcore_barrier()
        pltpu.sync_copy(
            acc_sh.at[pl.ds(row0, ROWS_PT)], out_hbm.at[c, pl.ds(row0, ROWS_PT)]
        )

    return agg


def _dinv_body(p_ref, o_ref):
    deg = p_ref[0] + p_ref[1] + 1.0
    o_ref[...] = lax.rsqrt(deg)


def _mm1_body(x_ref, w_ref, dinv_ref, o_ref):
    h = lax.dot_general(
        x_ref[...], w_ref[...], (((1,), (1,)), ((), ())),
        preferred_element_type=jnp.float32,
    )
    o_ref[...] = h * dinv_ref[...]


def _mid_body(acc_ref, g1_ref, dinv_ref, b1_ref, w2_ref, o_ref):
    i = pl.program_id(0)
    dinv = dinv_ref[...]
    u = dinv * (acc_ref[0] + acc_ref[1] + g1_ref[...]) + b1_ref[...]
    h = jnp.maximum(u, 0.0)
    rid = lax.broadcasted_iota(jnp.int32, (BLK, 1), 0) + i * BLK
    h = jnp.where(rid < NNODES, h, 0.0)
    g2 = lax.dot_general(
        h, w2_ref[...], (((1,), (1,)), ((), ())),
        preferred_element_type=jnp.float32,
    )
    o_ref[...] = g2 * dinv


def _fin_body(acc_ref, g2_ref, dinv_ref, b2_ref, wfc_ref, bfc_ref,
              n_ref, grep_ref, out_ref, sacc):
    i = pl.program_id(0)
    dinv = dinv_ref[...]
    u = dinv * (acc_ref[0][:, :64] + acc_ref[1][:, :64] + g2_ref[:, :64]) \
        + b2_ref[...]
    h = jnp.maximum(u, 0.0)
    rid = lax.broadcasted_iota(jnp.int32, (BLK, 1), 0) + i * BLK
    h = jnp.where(rid < NNODES, h, 0.0)
    n_ref[...] = h
    bs = jnp.sum(h, axis=0, keepdims=True)

    @pl.when(i == 0)
    def _():
        sacc[...] = bs

    @pl.when(i > 0)
    def _():
        sacc[...] = sacc[...] + bs

    @pl.when(i == GRID - 1)
    def _():
        grep = sacc[...] * (1.0 / NNODES)
        grep_ref[...] = grep
        logits = lax.dot_general(
            grep, wfc_ref[...], (((1,), (1,)), ((), ())),
            preferred_element_type=jnp.float32,
        ) + bfc_ref[...]
        m = jnp.max(logits, axis=1, keepdims=True)
        e = jnp.exp(logits - m)
        out_ref[...] = e / jnp.sum(e, axis=1, keepdims=True)


def kernel(x, edge_index, W1, b1, W2, b2, Wfc, bfc):
    f32 = jnp.float32
    x_pad = jnp.pad(x, ((0, NPAD - NNODES), (0, 0)))
    src3 = jnp.pad(edge_index[0], (0, EPAD - NEDGES),
                   constant_values=NNODES).reshape(NW, NCH, CH)
    dst3 = jnp.pad(edge_index[1], (0, EPAD - NEDGES),
                   constant_values=NNODES).reshape(NW, NCH, CH)

    deg_parts = _make_deg()(dst3)
    p2 = deg_parts[:, :, 0].reshape(2, NPAD // 128, 128)

    dinv2 = pl.pallas_call(
        _dinv_body,
        out_shape=jax.ShapeDtypeStruct((NPAD // 128, 128), f32),
    )(p2)
    dinv_col = dinv2.reshape(NPAD, 1)

    g1 = pl.pallas_call(
        _mm1_body,
        grid=(GRID,),
        in_specs=[
            pl.BlockSpec((BLK, 128), lambda i: (i, 0)),
            pl.BlockSpec((128, 128), lambda i: (0, 0)),
            pl.BlockSpec((BLK, 1), lambda i: (i, 0)),
        ],
        out_specs=pl.BlockSpec((BLK, 128), lambda i: (i, 0)),
        out_shape=jax.ShapeDtypeStruct((NPAD, 128), f32),
    )(x_pad, W1, dinv_col)

    acc1 = _make_agg(128)(src3, dst3, g1)

    g2 = pl.pallas_call(
        _mid_body,
        grid=(GRID,),
        in_specs=[
            pl.BlockSpec((2, BLK, 128), lambda i: (0, i, 0)),
            pl.BlockSpec((BLK, 128), lambda i: (i, 0)),
            pl.BlockSpec((BLK, 1), lambda i: (i, 0)),
            pl.BlockSpec((1, 128), lambda i: (0, 0)),
            pl.BlockSpec((128, 128), lambda i: (0, 0)),
        ],
        out_specs=pl.BlockSpec((BLK, 128), lambda i: (i, 0)),
        out_shape=jax.ShapeDtypeStruct((NPAD, 128), f32),
    )(acc1, g1, dinv_col, b1.reshape(1, 128), jnp.pad(W2, ((0, 64), (0, 0))))

    acc2 = _make_agg(128)(src3, dst3, g2)

    n_pad, g_rep, out = pl.pallas_call(
        _fin_body,
        grid=(GRID,),
        in_specs=[
            pl.BlockSpec((2, BLK, 128), lambda i: (0, i, 0)),
            pl.BlockSpec((BLK, 128), lambda i: (i, 0)),
            pl.BlockSpec((BLK, 1), lambda i: (i, 0)),
            pl.BlockSpec((1, 64), lambda i: (0, 0)),
            pl.BlockSpec((2, 64), lambda i: (0, 0)),
            pl.BlockSpec((1, 2), lambda i: (0, 0)),
        ],
        out_specs=[
            pl.BlockSpec((BLK, 64), lambda i: (i, 0)),
            pl.BlockSpec((1, 64), lambda i: (0, 0)),
            pl.BlockSpec((1, 2), lambda i: (0, 0)),
        ],
        out_shape=[
            jax.ShapeDtypeStruct((NPAD, 64), f32),
            jax.ShapeDtypeStruct((1, 64), f32),
            jax.ShapeDtypeStruct((1, 2), f32),
        ],
        scratch_shapes=[pltpu.VMEM((1, 64), f32)],
    )(acc2, g2, dinv_col, b2.reshape(1, 64), Wfc, bfc.reshape(1, 2))

    return (out, n_pad[:NNODES], g_rep)


# speedup vs baseline: 1.5266x; 1.5266x over previous
import functools

import jax
import jax.numpy as jnp
from jax import lax
from jax.experimental import pallas as pl
from jax.experimental.pallas import tpu as pltpu
from jax.experimental.pallas import tpu_sc as plsc

NNODES = 10000
NPAD = 10240
NEDGES = 320000
CH = 128
NCH = 79
EPT = NCH * CH
NW = 32
EPAD = EPT * NW
ROWS_PT = NPAD // 16
BLK = 1024
GRID = NPAD // BLK


def _sc_mesh():
    return plsc.VectorSubcoreMesh(
        core_axis_name="c", subcore_axis_name="s", num_cores=2, num_subcores=16
    )


def _make_deg():
    @functools.partial(
        pl.kernel,
        out_type=jax.ShapeDtypeStruct((2, NPAD, 16), jnp.float32),
        mesh=_sc_mesh(),
        scratch_types=[
            pltpu.VMEM((NCH, CH), jnp.int32),
            pltpu.VMEM((CH, 16), jnp.float32),
            pltpu.VMEM((CH, 16), jnp.float32),
            pltpu.VMEM_SHARED((NPAD, 16), jnp.float32),
        ],
    )
    def deg(dst_hbm, out_hbm, dst_v, ones_v, zero_v, acc_sh):
        c = lax.axis_index("c")
        s = lax.axis_index("s")
        wid = s * 2 + c

        def fill(i, _):
            ones_v[i, pl.ds(0, 16)] = jnp.full((16,), 1.0, jnp.float32)
            zero_v[i, pl.ds(0, 16)] = jnp.zeros((16,), jnp.float32)
            return _

        lax.fori_loop(0, CH, fill, None)
        row0 = s * ROWS_PT
        for b in range(ROWS_PT // CH):
            pltpu.sync_copy(zero_v, acc_sh.at[pl.ds(row0 + b * CH, CH)])
        plsc.subcore_barrier()

        pltpu.sync_copy(dst_hbm.at[wid], dst_v)

        def body(j, _):
            pltpu.sync_copy(ones_v, acc_sh.at[dst_v.at[j]], add=True)
            return _

        lax.fori_loop(0, NCH, body, None)
        plsc.subcore_barrier()
        pltpu.sync_copy(
            acc_sh.at[pl.ds(row0, ROWS_PT)], out_hbm.at[c, pl.ds(row0, ROWS_PT)]
        )

    return deg


def _make_agg(D):
    @functools.partial(
        pl.kernel,
        out_type=jax.ShapeDtypeStruct((2, NPAD, D), jnp.float32),
        mesh=_sc_mesh(),
        scratch_types=[
            pltpu.VMEM((NCH, CH), jnp.int32),
            pltpu.VMEM((NCH, CH), jnp.int32),
            pltpu.VMEM((CH, D), jnp.float32),
            pltpu.VMEM_SHARED((NPAD, D), jnp.float32),
            pltpu.SemaphoreType.DMA,
        ],
    )
    def agg(src_hbm, dst_hbm, table_hbm, out_hbm, src_v, dst_v, rows_v,
            acc_sh, sem):
        c = lax.axis_index("c")
        s = lax.axis_index("s")
        wid = s * 2 + c

        def zrow(i, _):
            for k in range(D // 16):
                rows_v[i, pl.ds(k * 16, 16)] = jnp.zeros((16,), jnp.float32)
            return _

        lax.fori_loop(0, CH, zrow, None)
        row0 = s * ROWS_PT
        for b in range(ROWS_PT // CH):
            pltpu.sync_copy(rows_v, acc_sh.at[pl.ds(row0 + b * CH, CH)])
        plsc.subcore_barrier()

        pltpu.sync_copy(src_hbm.at[wid], src_v)
        pltpu.sync_copy(dst_hbm.at[wid], dst_v)

        def body(j, _):
            pltpu.async_copy(table_hbm.at[src_v.at[j]], rows_v, sem).wait()
            pltpu.sync_copy(rows_v, acc_sh.at[dst_v.at[j]], add=True)
            return _

        lax.fori_loop(0, NCH, body, None)
        plsc.subcore_barrier()
        pltpu.sync_copy(
            acc_sh.at[pl.ds(row0, ROWS_PT)], out_hbm.at[c, pl.ds(row0, ROWS_PT)]
        )

    return agg


def _dinv_body(p_ref, o_ref):
    deg = p_ref[0] + p_ref[1] + 1.0
    o_ref[...] = lax.rsqrt(deg)


def _mm1_body(x_ref, w_ref, dinv_ref, o_ref):
    h = lax.dot_general(
        x_ref[...], w_ref[...], (((1,), (1,)), ((), ())),
        preferred_element_type=jnp.float32,
    )
    o_ref[...] = h * dinv_ref[...]


def _mid_body(acc_ref, g1_ref, dinv_ref, b1_ref, w2_ref, o_ref):
    i = pl.program_id(0)
    dinv = dinv_ref[...]
    u = dinv * (acc_ref[0] + acc_ref[1] + g1_ref[...]) + b1_ref[...]
    h = jnp.maximum(u, 0.0)
    rid = lax.broadcasted_iota(jnp.int32, (BLK, 1), 0) + i * BLK
    h = jnp.where(rid < NNODES, h, 0.0)
    g2 = lax.dot_general(
        h, w2_ref[...], (((1,), (1,)), ((), ())),
        preferred_element_type=jnp.float32,
    )
    o_ref[...] = g2 * dinv


def _fin_body(acc_ref, g2_ref, dinv_ref, b2_ref, wfc_ref, bfc_ref,
              n_ref, grep_ref, out_ref, sacc):
    i = pl.program_id(0)
    dinv = dinv_ref[...]
    u = dinv * (acc_ref[0][:, :64] + acc_ref[1][:, :64] + g2_ref[:, :64]) \
        + b2_ref[...]
    h = jnp.maximum(u, 0.0)
    rid = lax.broadcasted_iota(jnp.int32, (BLK, 1), 0) + i * BLK
    h = jnp.where(rid < NNODES, h, 0.0)
    n_ref[...] = h
    bs = jnp.sum(h, axis=0, keepdims=True)

    @pl.when(i == 0)
    def _():
        sacc[...] = bs

    @pl.when(i > 0)
    def _():
        sacc[...] = sacc[...] + bs

    @pl.when(i == GRID - 1)
    def _():
        grep = sacc[...] * (1.0 / NNODES)
        grep_ref[...] = grep
        logits = lax.dot_general(
            grep, wfc_ref[...], (((1,), (1,)), ((), ())),
            preferred_element_type=jnp.float32,
        ) + bfc_ref[...]
        m = jnp.max(logits, axis=1, keepdims=True)
        e = jnp.exp(logits - m)
        out_ref[...] = e / jnp.sum(e, axis=1, keepdims=True)


def kernel(x, edge_index, W1, b1, W2, b2, Wfc, bfc):
    f32 = jnp.float32
    x_pad = jnp.pad(x, ((0, NPAD - NNODES), (0, 0)))
    npad_e = EPAD - NEDGES
    pad_dst = NNODES + 1 + (jnp.arange(npad_e, dtype=jnp.int32) % 239)
    src3 = jnp.pad(edge_index[0], (0, npad_e),
                   constant_values=NNODES).reshape(NW, NCH, CH)
    dst3 = jnp.concatenate([edge_index[1], pad_dst]).reshape(NW, NCH, CH)

    deg_parts = _make_deg()(dst3)
    p2 = deg_parts[:, :, 0].reshape(2, NPAD // 128, 128)

    dinv2 = pl.pallas_call(
        _dinv_body,
        out_shape=jax.ShapeDtypeStruct((NPAD // 128, 128), f32),
    )(p2)
    dinv_col = dinv2.reshape(NPAD, 1)

    g1 = pl.pallas_call(
        _mm1_body,
        grid=(GRID,),
        in_specs=[
            pl.BlockSpec((BLK, 128), lambda i: (i, 0)),
            pl.BlockSpec((128, 128), lambda i: (0, 0)),
            pl.BlockSpec((BLK, 1), lambda i: (i, 0)),
        ],
        out_specs=pl.BlockSpec((BLK, 128), lambda i: (i, 0)),
        out_shape=jax.ShapeDtypeStruct((NPAD, 128), f32),
    )(x_pad, W1, dinv_col)

    acc1 = _make_agg(128)(src3, dst3, g1)

    g2 = pl.pallas_call(
        _mid_body,
        grid=(GRID,),
        in_specs=[
            pl.BlockSpec((2, BLK, 128), lambda i: (0, i, 0)),
            pl.BlockSpec((BLK, 128), lambda i: (i, 0)),
            pl.BlockSpec((BLK, 1), lambda i: (i, 0)),
            pl.BlockSpec((1, 128), lambda i: (0, 0)),
            pl.BlockSpec((128, 128), lambda i: (0, 0)),
        ],
        out_specs=pl.BlockSpec((BLK, 128), lambda i: (i, 0)),
        out_shape=jax.ShapeDtypeStruct((NPAD, 128), f32),
    )(acc1, g1, dinv_col, b1.reshape(1, 128), jnp.pad(W2, ((0, 64), (0, 0))))

    acc2 = _make_agg(128)(src3, dst3, g2)

    n_pad, g_rep, out = pl.pallas_call(
        _fin_body,
        grid=(GRID,),
        in_specs=[
            pl.BlockSpec((2, BLK, 128), lambda i: (0, i, 0)),
            pl.BlockSpec((BLK, 128), lambda i: (i, 0)),
            pl.BlockSpec((BLK, 1), lambda i: (i, 0)),
            pl.BlockSpec((1, 64), lambda i: (0, 0)),
            pl.BlockSpec((2, 64), lambda i: (0, 0)),
            pl.BlockSpec((1, 2), lambda i: (0, 0)),
        ],
        out_specs=[
            pl.BlockSpec((BLK, 64), lambda i: (i, 0)),
            pl.BlockSpec((1, 64), lambda i: (0, 0)),
            pl.BlockSpec((1, 2), lambda i: (0, 0)),
        ],
        out_shape=[
            jax.ShapeDtypeStruct((NPAD, 64), f32),
            jax.ShapeDtypeStruct((1, 64), f32),
            jax.ShapeDtypeStruct((1, 2), f32),
        ],
        scratch_shapes=[pltpu.VMEM((1, 64), f32)],
    )(acc2, g2, dinv_col, b2.reshape(1, 64), Wfc, bfc.reshape(1, 2))

    return (out, n_pad[:NNODES], g_rep)


# speedup vs baseline: 1.6767x; 1.0983x over previous
import functools

import jax
import jax.numpy as jnp
from jax import lax
from jax.experimental import pallas as pl
from jax.experimental.pallas import tpu as pltpu
from jax.experimental.pallas import tpu_sc as plsc

NNODES = 10000
NPAD = 10240
NEDGES = 320000
CH = 128
NCH = 79
EPT = NCH * CH
NW = 32
EPAD = EPT * NW
ROWS_PT = NPAD // 16
BLK = 1024
GRID = NPAD // BLK


def _sc_mesh():
    return plsc.VectorSubcoreMesh(
        core_axis_name="c", subcore_axis_name="s", num_cores=2, num_subcores=16
    )


def _make_deg():
    @functools.partial(
        pl.kernel,
        out_type=jax.ShapeDtypeStruct((NW, NPAD), jnp.float32),
        mesh=_sc_mesh(),
        scratch_types=[
            pltpu.VMEM((NCH, CH), jnp.int32),
            pltpu.VMEM((NPAD,), jnp.float32),
        ],
        compiler_params=pltpu.CompilerParams(needs_layout_passes=False),
    )
    def deg(dst_hbm, out_hbm, dst_v, bins_v):
        c = lax.axis_index("c")
        s = lax.axis_index("s")
        wid = s * 2 + c

        def zero(i, _):
            bins_v[pl.ds(i * 16, 16)] = jnp.zeros((16,), jnp.float32)
            return _

        lax.fori_loop(0, NPAD // 16, zero, None)
        pltpu.sync_copy(dst_hbm.at[wid], dst_v)
        ones16 = jnp.full((16,), 1.0, jnp.float32)

        def body(j, _):
            for k in range(CH // 16):
                idx = dst_v[j, pl.ds(k * 16, 16)]
                plsc.addupdate_scatter(bins_v, [idx], ones16)
            return _

        lax.fori_loop(0, NCH, body, None)
        pltpu.sync_copy(bins_v, out_hbm.at[wid])

    return deg


def _make_agg(D):
    @functools.partial(
        pl.kernel,
        out_type=jax.ShapeDtypeStruct((2, NPAD, D), jnp.float32),
        mesh=_sc_mesh(),
        scratch_types=[
            pltpu.VMEM((NCH, CH), jnp.int32),
            pltpu.VMEM((NCH, CH), jnp.int32),
            pltpu.VMEM((CH, D), jnp.float32),
            pltpu.VMEM_SHARED((NPAD, D), jnp.float32),
            pltpu.SemaphoreType.DMA,
        ],
    )
    def agg(src_hbm, dst_hbm, table_hbm, out_hbm, src_v, dst_v, rows_v,
            acc_sh, sem):
        c = lax.axis_index("c")
        s = lax.axis_index("s")
        wid = s * 2 + c

        def zrow(i, _):
            for k in range(D // 16):
                rows_v[i, pl.ds(k * 16, 16)] = jnp.zeros((16,), jnp.float32)
            return _

        lax.fori_loop(0, CH, zrow, None)
        row0 = s * ROWS_PT
        for b in range(ROWS_PT // CH):
            pltpu.sync_copy(rows_v, acc_sh.at[pl.ds(row0 + b * CH, CH)])
        plsc.subcore_barrier()

        pltpu.sync_copy(src_hbm.at[wid], src_v)
        pltpu.sync_copy(dst_hbm.at[wid], dst_v)

        def body(j, _):
            pltpu.async_copy(table_hbm.at[src_v.at[j]], rows_v, sem).wait()
            pltpu.sync_copy(rows_v, acc_sh.at[dst_v.at[j]], add=True)
            return _

        lax.fori_loop(0, NCH, body, None)
        plsc.subcore_barrier()
        pltpu.sync_copy(
            acc_sh.at[pl.ds(row0, ROWS_PT)], out_hbm.at[c, pl.ds(row0, ROWS_PT)]
        )

    return agg


def _dinv_body(p_ref, o_ref):
    deg = jnp.sum(p_ref[...], axis=0) + 1.0
    o_ref[...] = lax.rsqrt(deg)


def _mm1_body(x_ref, w_ref, dinv_ref, o_ref):
    h = lax.dot_general(
        x_ref[...], w_ref[...], (((1,), (1,)), ((), ())),
        preferred_element_type=jnp.float32,
    )
    o_ref[...] = h * dinv_ref[...]


def _mid_body(acc_ref, g1_ref, dinv_ref, b1_ref, w2_ref, o_ref):
    i = pl.program_id(0)
    dinv = dinv_ref[...]
    u = dinv * (acc_ref[0] + acc_ref[1] + g1_ref[...]) + b1_ref[...]
    h = jnp.maximum(u, 0.0)
    rid = lax.broadcasted_iota(jnp.int32, (BLK, 1), 0) + i * BLK
    h = jnp.where(rid < NNODES, h, 0.0)
    g2 = lax.dot_general(
        h, w2_ref[...], (((1,), (1,)), ((), ())),
        preferred_element_type=jnp.float32,
    )
    o_ref[...] = g2 * dinv


def _fin_body(acc_ref, g2_ref, dinv_ref, b2_ref, wfc_ref, bfc_ref,
              n_ref, grep_ref, out_ref, sacc):
    i = pl.program_id(0)
    dinv = dinv_ref[...]
    u = dinv * (acc_ref[0][:, :64] + acc_ref[1][:, :64] + g2_ref[:, :64]) \
        + b2_ref[...]
    h = jnp.maximum(u, 0.0)
    rid = lax.broadcasted_iota(jnp.int32, (BLK, 1), 0) + i * BLK
    h = jnp.where(rid < NNODES, h, 0.0)
    n_ref[...] = h
    bs = jnp.sum(h, axis=0, keepdims=True)

    @pl.when(i == 0)
    def _():
        sacc[...] = bs

    @pl.when(i > 0)
    def _():
        sacc[...] = sacc[...] + bs

    @pl.when(i == GRID - 1)
    def _():
        grep = sacc[...] * (1.0 / NNODES)
        grep_ref[...] = grep
        logits = lax.dot_general(
            grep, wfc_ref[...], (((1,), (1,)), ((), ())),
            preferred_element_type=jnp.float32,
        ) + bfc_ref[...]
        m = jnp.max(logits, axis=1, keepdims=True)
        e = jnp.exp(logits - m)
        out_ref[...] = e / jnp.sum(e, axis=1, keepdims=True)


def kernel(x, edge_index, W1, b1, W2, b2, Wfc, bfc):
    f32 = jnp.float32
    x_pad = jnp.pad(x, ((0, NPAD - NNODES), (0, 0)))
    npad_e = EPAD - NEDGES
    pad_dst = NNODES + 1 + (jnp.arange(npad_e, dtype=jnp.int32) % 239)
    src3 = jnp.pad(edge_index[0], (0, npad_e),
                   constant_values=NNODES).reshape(NW, NCH, CH)
    dst3 = jnp.concatenate([edge_index[1], pad_dst]).reshape(NW, NCH, CH)

    deg_parts = _make_deg()(dst3)
    p2 = deg_parts.reshape(NW, NPAD // 128, 128)

    dinv2 = pl.pallas_call(
        _dinv_body,
        out_shape=jax.ShapeDtypeStruct((NPAD // 128, 128), f32),
    )(p2)
    dinv_col = dinv2.reshape(NPAD, 1)

    g1 = pl.pallas_call(
        _mm1_body,
        grid=(GRID,),
        in_specs=[
            pl.BlockSpec((BLK, 128), lambda i: (i, 0)),
            pl.BlockSpec((128, 128), lambda i: (0, 0)),
            pl.BlockSpec((BLK, 1), lambda i: (i, 0)),
        ],
        out_specs=pl.BlockSpec((BLK, 128), lambda i: (i, 0)),
        out_shape=jax.ShapeDtypeStruct((NPAD, 128), f32),
    )(x_pad, W1, dinv_col)

    acc1 = _make_agg(128)(src3, dst3, g1)

    g2 = pl.pallas_call(
        _mid_body,
        grid=(GRID,),
        in_specs=[
            pl.BlockSpec((2, BLK, 128), lambda i: (0, i, 0)),
            pl.BlockSpec((BLK, 128), lambda i: (i, 0)),
            pl.BlockSpec((BLK, 1), lambda i: (i, 0)),
            pl.BlockSpec((1, 128), lambda i: (0, 0)),
            pl.BlockSpec((128, 128), lambda i: (0, 0)),
        ],
        out_specs=pl.BlockSpec((BLK, 128), lambda i: (i, 0)),
        out_shape=jax.ShapeDtypeStruct((NPAD, 128), f32),
    )(acc1, g1, dinv_col, b1.reshape(1, 128), jnp.pad(W2, ((0, 64), (0, 0))))

    acc2 = _make_agg(128)(src3, dst3, g2)

    n_pad, g_rep, out = pl.pallas_call(
        _fin_body,
        grid=(GRID,),
        in_specs=[
            pl.BlockSpec((2, BLK, 128), lambda i: (0, i, 0)),
            pl.BlockSpec((BLK, 128), lambda i: (i, 0)),
            pl.BlockSpec((BLK, 1), lambda i: (i, 0)),
            pl.BlockSpec((1, 64), lambda i: (0, 0)),
            pl.BlockSpec((2, 64), lambda i: (0, 0)),
            pl.BlockSpec((1, 2), lambda i: (0, 0)),
        ],
        out_specs=[
            pl.BlockSpec((BLK, 64), lambda i: (i, 0)),
            pl.BlockSpec((1, 64), lambda i: (0, 0)),
            pl.BlockSpec((1, 2), lambda i: (0, 0)),
        ],
        out_shape=[
            jax.ShapeDtypeStruct((NPAD, 64), f32),
            jax.ShapeDtypeStruct((1, 64), f32),
            jax.ShapeDtypeStruct((1, 2), f32),
        ],
        scratch_shapes=[pltpu.VMEM((1, 64), f32)],
    )(acc2, g2, dinv_col, b2.reshape(1, 64), Wfc, bfc.reshape(1, 2))

    return (out, n_pad[:NNODES], g_rep)
